# baseline (speedup 1.0000x reference)
import functools

import jax
import jax.numpy as jnp
from jax import lax
from jax.experimental import pallas as pl
from jax.experimental.pallas import tpu as pltpu
from jax.experimental.pallas import tpu_sc as plsc

N = 100000
C = 128
NC = 2
NS = 16
NW = NC * NS
RPW = N // NW
CH = 125
NCHUNK = RPW // CH
CHW = CH * C
GROUPS = 8
TAIL = CH - (GROUPS - 1) * 16
IDXWIN = 3136
XPAD = 100008


@functools.partial(
    pl.kernel,
    out_type=jax.ShapeDtypeStruct((N * C,), jnp.float32),
    mesh=plsc.VectorSubcoreMesh(core_axis_name="c", subcore_axis_name="s"),
    compiler_params=pltpu.CompilerParams(needs_layout_passes=False),
    scratch_types=[
        pltpu.VMEM((IDXWIN,), jnp.int32),
        pltpu.VMEM((2 * CHW,), jnp.float32),
        pltpu.SemaphoreType.DMA,
        pltpu.SemaphoreType.DMA,
    ],
)
def _onehot_sc(x_hbm, out_hbm, idxbuf, buf, sem0, sem1):
    w = lax.axis_index("s") * NC + lax.axis_index("c")
    base_row = w * RPW
    win = (base_row // 8) * 8
    off = base_row - win

    pltpu.sync_copy(x_hbm.at[pl.ds(win, IDXWIN)], idxbuf)

    iota = lax.broadcasted_iota(jnp.int32, (16,), 0)
    zeros = jnp.zeros((16,), jnp.float32)
    ones = jnp.ones((16,), jnp.float32)
    colv = iota * C
    tail_mask = iota < TAIL

    def _zbody(i, carry):
        base = i * 1280
        for k in range(80):
            plsc.store_scatter(buf, [base + k * 16 + iota], zeros)
        return carry

    lax.fori_loop(0, 2 * CHW // 1280, _zbody, 0)

    sems = (sem0, sem1)

    def _scatter(c, slot_base, vals):
        for j in range(GROUPS):
            iv = plsc.load_gather(idxbuf, [off + (c * CH + j * 16) + iota])
            flat = iv + (colv + slot_base + j * 16 * C)
            if j < GROUPS - 1:
                plsc.store_scatter(buf, [flat], vals)
            else:
                plsc.store_scatter(buf, [flat], vals, mask=tail_mask)

    for c in range(NCHUNK):
        b = c & 1
        sb = b * CHW
        if c >= 2:
            pltpu.make_async_copy(
                buf.at[pl.ds(sb, CHW)],
                out_hbm.at[pl.ds(base_row * C + (c - 2) * CHW, CHW)],
                sems[b],
            ).wait()
            _scatter(c - 2, sb, zeros)
        _scatter(c, sb, ones)
        pltpu.make_async_copy(
            buf.at[pl.ds(sb, CHW)],
            out_hbm.at[pl.ds(base_row * C + c * CHW, CHW)],
            sems[b],
        ).start()

    for b, last_c in ((1, NCHUNK - 2), (0, NCHUNK - 1)):
        pltpu.make_async_copy(
            buf.at[pl.ds(b * CHW, CHW)],
            out_hbm.at[pl.ds(base_row * C + last_c * CHW, CHW)],
            sems[b],
        ).wait()


def kernel(x):
    xp = jnp.pad(x, (0, XPAD - N))
    embd = _onehot_sc(xp).reshape(N, C)
    return (embd, embd, x)


# speedup vs baseline: 1.2481x; 1.2481x over previous
import functools

import jax
import jax.numpy as jnp
from jax import lax
from jax.experimental import pallas as pl
from jax.experimental.pallas import tpu as pltpu
from jax.experimental.pallas import tpu_sc as plsc

N = 100000
C = 128
NC = 2
NS = 16
NW = NC * NS
RPW = N // NW
CH = 125
NCHUNK = RPW // CH
CHW = CH * C
GROUPS = 8
TAIL = CH - (GROUPS - 1) * 16
IDXWIN = 3136
XPAD = 100008


@functools.partial(
    pl.kernel,
    out_type=(
        jax.ShapeDtypeStruct((N * C,), jnp.float32),
        jax.ShapeDtypeStruct((N * C,), jnp.float32),
    ),
    mesh=plsc.VectorSubcoreMesh(core_axis_name="c", subcore_axis_name="s"),
    compiler_params=pltpu.CompilerParams(needs_layout_passes=False),
    scratch_types=[
        pltpu.VMEM((IDXWIN,), jnp.int32),
        pltpu.VMEM((2 * CHW,), jnp.float32),
        pltpu.SemaphoreType.DMA,
        pltpu.SemaphoreType.DMA,
    ],
)
def _onehot_sc(x_hbm, out_hbm, out2_hbm, idxbuf, buf, sem0, sem1):
    w = lax.axis_index("s") * NC + lax.axis_index("c")
    base_row = w * RPW
    win = (base_row // 8) * 8
    off = base_row - win

    pltpu.sync_copy(x_hbm.at[pl.ds(win, IDXWIN)], idxbuf)

    iota = lax.broadcasted_iota(jnp.int32, (16,), 0)
    zeros = jnp.zeros((16,), jnp.float32)
    ones = jnp.ones((16,), jnp.float32)
    colv = iota * C
    tail_mask = iota < TAIL

    def _zbody(i, carry):
        base = i * 1280
        for k in range(80):
            plsc.store_scatter(buf, [base + k * 16 + iota], zeros)
        return carry

    lax.fori_loop(0, 2 * CHW // 1280, _zbody, 0)

    sems = (sem0, sem1)

    def _scatter(c, slot_base, vals):
        for j in range(GROUPS):
            iv = plsc.load_gather(idxbuf, [off + (c * CH + j * 16) + iota])
            flat = iv + (colv + slot_base + j * 16 * C)
            if j < GROUPS - 1:
                plsc.store_scatter(buf, [flat], vals)
            else:
                plsc.store_scatter(buf, [flat], vals, mask=tail_mask)

    for c in range(NCHUNK):
        b = c & 1
        sb = b * CHW
        if c >= 2:
            for dst in (out_hbm, out2_hbm):
                pltpu.make_async_copy(
                    buf.at[pl.ds(sb, CHW)],
                    dst.at[pl.ds(base_row * C + (c - 2) * CHW, CHW)],
                    sems[b],
                ).wait()
            _scatter(c - 2, sb, zeros)
        _scatter(c, sb, ones)
        for dst in (out_hbm, out2_hbm):
            pltpu.make_async_copy(
                buf.at[pl.ds(sb, CHW)],
                dst.at[pl.ds(base_row * C + c * CHW, CHW)],
                sems[b],
            ).start()

    for b, last_c in ((1, NCHUNK - 2), (0, NCHUNK - 1)):
        for dst in (out_hbm, out2_hbm):
            pltpu.make_async_copy(
                buf.at[pl.ds(b * CHW, CHW)],
                dst.at[pl.ds(base_row * C + last_c * CHW, CHW)],
                sems[b],
            ).wait()


def kernel(x):
    xp = jnp.pad(x, (0, XPAD - N))
    e1, e2 = _onehot_sc(xp)
    return (e1.reshape(N, C), e2.reshape(N, C), x)
